# Initial kernel scaffold; baseline (speedup 1.0000x reference)
import jax
import jax.numpy as jnp
from jax.experimental import pallas as pl


def kernel(y, out, w, total_size):
    raise NotImplementedError("write your pallas kernel here")


# speedup vs baseline: 12.3631x; 12.3631x over previous
import functools

import jax
import jax.numpy as jnp
from jax import lax
from jax.experimental import pallas as pl
from jax.experimental.pallas import tpu as pltpu
from jax.experimental.pallas import tpu_sc as plsc

_NUM_CLASSES = 8
_NEG_POS_RATIO = 3

_B, _H, _W, _C = 4, 512, 512, _NUM_CLASSES
_N = _H * _W * _C
_NCORE = 2
_NSUB = 16
_BPC = _B // _NCORE
_NSL = _N // _NSUB
_CH = 8192
_NCHUNK = _NSL // _CH
_VPC = _CH // 16
_POS_ROW = 128
_MSE_ROW = 129


def _sc_body(y_hbm, o_hbm, w_hbm, ts_hbm, out_hbm, neg_hbm,
             ybuf, obuf, wbuf, negbuf, cnt_h, sum_h, gcnt, gsum,
             tsbuf, outbuf, idx_lo, idx_hi, sh_cnt, sh_sum):
    core = lax.axis_index("c")
    sid = lax.axis_index("s")
    lanes = lax.iota(jnp.int32, 16)
    onesf = jnp.full((16,), 1.0, jnp.float32)
    zerof = jnp.zeros((16,), jnp.float32)

    def init_idx(i, _):
        v = lanes + i * 16
        idx_lo[pl.ds(i * 16, 16)] = v
        idx_hi[pl.ds(i * 16, 16)] = v + 128
        return 0

    lax.fori_loop(0, 8, init_idx, 0)
    pltpu.sync_copy(ts_hbm, tsbuf)

    def zero_local(i, _):
        cnt_h[i] = zerof
        sum_h[i] = zerof
        return 0

    def reset_hists():
        plsc.subcore_barrier()
        lax.fori_loop(0, 256, zero_local, 0)

        @pl.when(sid == 0)
        def _():
            pltpu.sync_copy(cnt_h, sh_cnt)
            pltpu.sync_copy(sum_h, sh_sum)

        plsc.subcore_barrier()

    def combine_hists():
        pltpu.sync_copy(cnt_h.at[pl.ds(0, 128)], sh_cnt.at[idx_lo], add=True)
        pltpu.sync_copy(cnt_h.at[pl.ds(128, 128)], sh_cnt.at[idx_hi], add=True)
        pltpu.sync_copy(sum_h.at[pl.ds(0, 128)], sh_sum.at[idx_lo], add=True)
        pltpu.sync_copy(sum_h.at[pl.ds(128, 128)], sh_sum.at[idx_hi], add=True)
        plsc.subcore_barrier()
        pltpu.sync_copy(sh_cnt, gcnt)
        pltpu.sync_copy(sh_sum, gsum)

    def search(kk, top_bucket):
        def body(j, carry):
            cum, above, t, r, cnt_t, sum_t, found = carry
            b = top_bucket - 1 - j
            cj = jnp.sum(gcnt[b])
            sj = jnp.sum(gsum[b])
            here = jnp.logical_and(jnp.logical_not(found), cum + cj >= kk)
            t = jnp.where(here, b, t)
            r = jnp.where(here, kk - cum, r)
            cnt_t = jnp.where(here, cj, cnt_t)
            sum_t = jnp.where(here, sj, sum_t)
            above = jnp.where(jnp.logical_or(found, here), above, above + sj)
            return (cum + cj, above, t, r, cnt_t, sum_t,
                    jnp.logical_or(found, here))

        init = (jnp.float32(0), jnp.float32(0), jnp.int32(-1), jnp.float32(0),
                jnp.float32(0), jnp.float32(0), False)
        _, above, t, r, cnt_t, sum_t, _ = lax.fori_loop(
            0, top_bucket, body, init)
        return above, t, r, cnt_t, sum_t

    def sweep1(base):
        def chunk_body(ci, carry):
            pos_a, mse_a = carry
            off = base + ci * _CH
            pltpu.sync_copy(y_hbm.at[pl.ds(off, _CH)], ybuf)
            pltpu.sync_copy(o_hbm.at[pl.ds(off, _CH)], obuf)
            pltpu.sync_copy(w_hbm.at[pl.ds(off, _CH)], wbuf)

            def vbody(vi, c2):
                pa, ma = c2
                sl = pl.ds(vi * 16, 16)
                yv = ybuf[sl]
                ov = obuf[sl]
                wv = wbuf[sl]
                d = ov - yv
                m = d * d
                ma = ma + m
                posm = wv > 0
                pa = pa + jnp.where(posm, wv * m, 0.0)
                negm = jnp.logical_and(ov > 0, jnp.logical_not(posm))
                nv = jnp.where(negm, m, 0.0)
                negbuf[sl] = nv
                bits = plsc.bitcast(nv, jnp.int32)
                b1 = bits >> 24
                plsc.addupdate_scatter(cnt_h, [b1, lanes], onesf)
                plsc.addupdate_scatter(sum_h, [b1, lanes], nv)
                return (pa, ma)

            pos_a, mse_a = lax.fori_loop(0, _VPC, vbody, (pos_a, mse_a))
            pltpu.sync_copy(negbuf, neg_hbm.at[pl.ds(off, _CH)])
            return (pos_a, mse_a)

        pos_a, mse_a = lax.fori_loop(0, _NCHUNK, chunk_body, (zerof, zerof))
        cnt_h[_POS_ROW] = pos_a
        cnt_h[_MSE_ROW] = mse_a

    def sweep_refine(base, shift, prev_shift, prefix):
        def chunk_body(ci, _):
            off = base + ci * _CH
            pltpu.sync_copy(neg_hbm.at[pl.ds(off, _CH)], negbuf)

            def vbody(vi, __):
                sl = pl.ds(vi * 16, 16)
                nv = negbuf[sl]
                bits = plsc.bitcast(nv, jnp.int32)
                sel = (bits >> prev_shift) == prefix
                bb = (bits >> shift) & 0xFF
                plsc.addupdate_scatter(cnt_h, [bb, lanes], onesf, mask=sel)
                plsc.addupdate_scatter(sum_h, [bb, lanes], nv, mask=sel)
                return 0

            lax.fori_loop(0, _VPC, vbody, 0)
            return 0

        lax.fori_loop(0, _NCHUNK, chunk_body, 0)

    kvec = jnp.minimum(tsbuf[...].astype(jnp.int32) * _NEG_POS_RATIO,
                       _N).astype(jnp.float32)
    loss_acc = zerof
    mse_acc = zerof
    for bi in range(_BPC):
        b = core * _BPC + bi
        base = b * _N + sid * _NSL
        bsel = lanes == b
        kk = jnp.sum(jnp.where(bsel, kvec, 0.0))
        tsb = jnp.sum(jnp.where(bsel, tsbuf[...], 0.0))

        reset_hists()
        sweep1(base)
        combine_hists()
        above1, t1, r1, _, _ = search(kk, 128)
        pos_b = jnp.sum(gcnt[_POS_ROW])
        mse_b = jnp.sum(gcnt[_MSE_ROW])

        reset_hists()
        sweep_refine(base, 16, 24, t1)
        combine_hists()
        above2, t2, r2, _, _ = search(r1, 256)

        reset_hists()
        sweep_refine(base, 8, 16, (t1 << 8) | t2)
        combine_hists()
        above3, _, r3, cnt3, sum3 = search(r2, 256)

        mean3_v = jnp.full((16,), sum3) / jnp.maximum(jnp.full((16,), cnt3),
                                                      1.0)
        num_v = jnp.full((16,), pos_b + above1 + above2 + above3) \
            + r3 * mean3_v
        ts_v = jnp.full((16,), tsb)
        safe_ts = jnp.where(ts_v > 0, ts_v, 1.0)
        loss_acc = loss_acc + jnp.where(ts_v > 0, num_v / safe_ts, 0.0)
        mse_acc = mse_acc + mse_b

    plsc.subcore_barrier()

    @pl.when(sid == 0)
    def _():
        outv = jnp.where(lanes == 0, loss_acc,
                         jnp.where(lanes == 1, mse_acc, 0.0))
        outbuf[...] = outv
        pltpu.sync_copy(outbuf, out_hbm.at[core])


@functools.partial(jax.jit, static_argnames=())
def kernel(y, out, w, total_size):
    assert y.shape == (_B, _H, _W, _C)
    y2 = y.reshape(-1)
    o2 = out.reshape(-1)
    w2 = w.reshape(-1)
    ts_pad = jnp.zeros((16,), jnp.float32).at[:_B].set(
        total_size.reshape(-1).astype(jnp.float32))

    mesh = plsc.VectorSubcoreMesh(core_axis_name="c", subcore_axis_name="s",
                                  num_cores=_NCORE, num_subcores=_NSUB)
    fn = pl.kernel(
        _sc_body,
        out_type=(
            jax.ShapeDtypeStruct((_NCORE, 16), jnp.float32),
            jax.ShapeDtypeStruct((_B * _N,), jnp.float32),
        ),
        mesh=mesh,
        compiler_params=pltpu.CompilerParams(needs_layout_passes=False, use_tc_tiling_on_sc=False),
        scratch_types=[
            pltpu.VMEM((_CH,), jnp.float32),
            pltpu.VMEM((_CH,), jnp.float32),
            pltpu.VMEM((_CH,), jnp.float32),
            pltpu.VMEM((_CH,), jnp.float32),
            pltpu.VMEM((256, 16), jnp.float32),
            pltpu.VMEM((256, 16), jnp.float32),
            pltpu.VMEM((256, 16), jnp.float32),
            pltpu.VMEM((256, 16), jnp.float32),
            pltpu.VMEM((16,), jnp.float32),
            pltpu.VMEM((16,), jnp.float32),
            pltpu.VMEM((128,), jnp.int32),
            pltpu.VMEM((128,), jnp.int32),
            pltpu.VMEM_SHARED((256, 16), jnp.float32),
            pltpu.VMEM_SHARED((256, 16), jnp.float32),
        ],
    )
    partials, _neg = fn(y2, o2, w2, ts_pad)
    train_loss = (partials[0, 0] + partials[1, 0]) / _B
    mse_mean = (partials[0, 1] + partials[1, 1]) / (_B * _N)
    return ((train_loss + mse_mean) * 10).reshape(())
